# baseline (speedup 1.0000x reference)
import jax
import jax.numpy as jnp
from jax import lax
from jax.experimental import pallas as pl
from jax.experimental.pallas import tpu as pltpu
from jax.experimental.pallas import tpu_sc as plsc

_B = 16384
_D = 32
_NW = 32
_BPW = _B // _NW
_N = 1000000
_CHUNK = 128
_NCHUNK = _BPW // _CHUNK
_MU = 10000000.0 / (10000000.0 + 1000000.0 * 4.0)


def _mf_body(uid_hbm, iid_hbm, ue_hbm, ie_hbm, bu_hbm, bi_hbm, out_hbm,
             uid_v, iid_v, idx_u, idx_i, gu, gi, bu_v, bi_v, out_v, sem):
  wid = lax.axis_index("s") * 2 + lax.axis_index("c")
  base = wid * _BPW

  pltpu.sync_copy(uid_hbm.at[pl.ds(base, _BPW)], uid_v)
  pltpu.sync_copy(iid_hbm.at[pl.ds(base, _BPW)], iid_v)

  for c in range(_NCHUNK):
    sl = pl.ds(c * _CHUNK, _CHUNK)
    pltpu.async_copy(bu_hbm.at[uid_v.at[sl]], bu_v.at[sl], sem)
    pltpu.async_copy(bi_hbm.at[iid_v.at[sl]], bi_v.at[sl], sem)

  def build(g, carry):
    sl = pl.ds(g * 16, 16)
    u16 = uid_v[sl]
    i16 = iid_v[sl]
    c = g // (_CHUNK // 16)
    s = g % (_CHUNK // 16)
    dsl = pl.ds(s * 16, 16)
    for d in range(_D):
      idx_u[d * _NCHUNK + c, dsl] = u16 + d * _N
      idx_i[d * _NCHUNK + c, dsl] = i16 + d * _N
    return carry
  lax.fori_loop(0, _BPW // 16, build, 0)

  for row in range(_D * _NCHUNK):
    pltpu.async_copy(ue_hbm.at[idx_u.at[row]], gu.at[row], sem)
    pltpu.async_copy(ie_hbm.at[idx_i.at[row]], gi.at[row], sem)
  pltpu.make_async_copy(bu_hbm.at[pl.ds(0, _BPW)], bu_v, sem).wait()
  pltpu.make_async_copy(bi_hbm.at[pl.ds(0, _BPW)], bi_v, sem).wait()
  def drain(row, carry):
    pltpu.make_async_copy(ue_hbm.at[pl.ds(0, _CHUNK)], gu.at[row], sem).wait()
    pltpu.make_async_copy(ie_hbm.at[pl.ds(0, _CHUNK)], gi.at[row], sem).wait()
    return carry
  lax.fori_loop(0, _D * _NCHUNK, drain, 0)

  for c in range(_NCHUNK):
    def g_body(g, carry, c=c):
      off = g * 16
      goff = c * _CHUNK + off
      sl = pl.ds(off, 16)
      acc = bu_v[pl.ds(goff, 16)] + bi_v[pl.ds(goff, 16)] + _MU
      for d in range(_D):
        acc = acc + gu[d * _NCHUNK + c, sl] * gi[d * _NCHUNK + c, sl]
      out_v[pl.ds(goff, 16)] = acc
      return carry
    lax.fori_loop(0, _CHUNK // 16, g_body, 0)

  pltpu.sync_copy(out_v, out_hbm.at[pl.ds(base, _BPW)])


def kernel(x, user_embedding, item_embedding, b_u, b_i):
  uid = x[:, 0].astype(jnp.int32)
  iid = x[:, 1].astype(jnp.int32)
  ue_flat = user_embedding.T.reshape(-1)
  ie_flat = item_embedding.T.reshape(-1)
  mesh = plsc.VectorSubcoreMesh(core_axis_name="c", subcore_axis_name="s")
  mf = pl.kernel(
      _mf_body,
      out_type=jax.ShapeDtypeStruct((_B,), jnp.float32),
      mesh=mesh,
      scratch_types=[
          pltpu.VMEM((_BPW,), jnp.int32),
          pltpu.VMEM((_BPW,), jnp.int32),
          pltpu.VMEM((_D * _NCHUNK, _CHUNK), jnp.int32),
          pltpu.VMEM((_D * _NCHUNK, _CHUNK), jnp.int32),
          pltpu.VMEM((_D * _NCHUNK, _CHUNK), jnp.float32),
          pltpu.VMEM((_D * _NCHUNK, _CHUNK), jnp.float32),
          pltpu.VMEM((_BPW,), jnp.float32),
          pltpu.VMEM((_BPW,), jnp.float32),
          pltpu.VMEM((_BPW,), jnp.float32),
          pltpu.SemaphoreType.DMA,
      ],
  )
  return mf(uid, iid, ue_flat, ie_flat, b_u, b_i)


# speedup vs baseline: 5.7031x; 5.7031x over previous
import jax
import jax.numpy as jnp
from jax import lax
from jax.experimental import pallas as pl
from jax.experimental.pallas import tpu as pltpu
from jax.experimental.pallas import tpu_sc as plsc

_B = 16384
_D = 32
_NW = 32
_BPW = _B // _NW
_MU = 10000000.0 / (10000000.0 + 1000000.0 * 4.0)


def _mf_body(uid_hbm, iid_hbm, ue_hbm, ie_hbm, bu_hbm, bi_hbm, out_hbm,
             uid_v, iid_v, u_buf, i_buf, bu_v, bi_v, out_v, sem):
  wid = lax.axis_index("s") * 2 + lax.axis_index("c")
  base = wid * _BPW

  pltpu.sync_copy(uid_hbm.at[pl.ds(base, _BPW)], uid_v)
  pltpu.sync_copy(iid_hbm.at[pl.ds(base, _BPW)], iid_v)

  for j in range(_BPW // 128):
    sl = pl.ds(j * 128, 128)
    pltpu.async_copy(bu_hbm.at[uid_v.at[sl]], bu_v.at[sl], sem)
    pltpu.async_copy(bi_hbm.at[iid_v.at[sl]], bi_v.at[sl], sem)
  pltpu.make_async_copy(bu_hbm.at[pl.ds(0, _BPW)], bu_v, sem).wait()
  pltpu.make_async_copy(bi_hbm.at[pl.ds(0, _BPW)], bi_v, sem).wait()

  lane = lax.iota(jnp.int32, 16)
  bitrev = (((lane & 1) << 3) | ((lane & 2) << 1) |
            ((lane & 4) >> 1) | ((lane & 8) >> 3))

  def _perm(v, idx):
    return lax.gather(
        v, idx[:, None],
        lax.GatherDimensionNumbers(offset_dims=(), collapsed_slice_dims=(0,),
                                   start_index_map=(0,)),
        slice_sizes=(1,), mode=lax.GatherScatterMode.PROMISE_IN_BOUNDS)

  def _shuf(v, k):
    return _perm(v, lane ^ k)

  pp = _BPW // 2
  for p in range(2):
    pbase = p * pp

    def fire(g, carry, pbase=pbase):
      off = g * 16
      uvec = uid_v[pl.ds(pbase + off, 16)]
      ivec = iid_v[pl.ds(pbase + off, 16)]
      for r in range(16):
        uq = lax.shift_right_logical(uvec[r], 2)
        iq = lax.shift_right_logical(ivec[r], 2)
        pltpu.async_copy(ue_hbm.at[pl.ds(uq, 1)],
                         u_buf.at[pl.ds(off + r, 1)], sem)
        pltpu.async_copy(ie_hbm.at[pl.ds(iq, 1)],
                         i_buf.at[pl.ds(off + r, 1)], sem)
      return carry
    lax.fori_loop(0, pp // 16, fire, 0)
    pltpu.make_async_copy(ue_hbm.at[pl.ds(0, pp)], u_buf, sem).wait()
    pltpu.make_async_copy(ie_hbm.at[pl.ds(0, pp)], i_buf, sem).wait()

    def g_body(g, carry, pbase=pbase):
      off = g * 16
      uvec = uid_v[pl.ds(pbase + off, 16)]
      ivec = iid_v[pl.ds(pbase + off, 16)]
      vecs = []
      for r in range(16):
        row = off + r
        us = (uvec[r] & 3) * 32
        ist = (ivec[r] & 3) * 32
        vecs.append(
            u_buf[row, pl.ds(us, 16)] * i_buf[row, pl.ds(ist, 16)] +
            u_buf[row, pl.ds(us + 16, 16)] * i_buf[row, pl.ds(ist + 16, 16)])
      for k in (8, 4, 2, 1):
        m = (lane & k) == 0
        vecs = [jnp.where(m, x + _shuf(x, k), y + _shuf(y, k))
                for x, y in zip(vecs[0::2], vecs[1::2])]
      dots = _perm(vecs[0], bitrev)
      out_v[pl.ds(pbase + off, 16)] = (dots + bu_v[pl.ds(pbase + off, 16)] +
                                       bi_v[pl.ds(pbase + off, 16)] + _MU)
      return carry
    lax.fori_loop(0, pp // 16, g_body, 0)

  pltpu.sync_copy(out_v, out_hbm.at[pl.ds(base, _BPW)])


def kernel(x, user_embedding, item_embedding, b_u, b_i):
  uid = x[:, 0].astype(jnp.int32)
  iid = x[:, 1].astype(jnp.int32)
  ue_c = user_embedding.reshape(250000, 128)
  ie_c = item_embedding.reshape(250000, 128)
  mesh = plsc.VectorSubcoreMesh(core_axis_name="c", subcore_axis_name="s")
  mf = pl.kernel(
      _mf_body,
      out_type=jax.ShapeDtypeStruct((_B,), jnp.float32),
      mesh=mesh,
      scratch_types=[
          pltpu.VMEM((_BPW,), jnp.int32),
          pltpu.VMEM((_BPW,), jnp.int32),
          pltpu.VMEM((_BPW // 2, 128), jnp.float32),
          pltpu.VMEM((_BPW // 2, 128), jnp.float32),
          pltpu.VMEM((_BPW,), jnp.float32),
          pltpu.VMEM((_BPW,), jnp.float32),
          pltpu.VMEM((_BPW,), jnp.float32),
          pltpu.SemaphoreType.DMA,
      ],
  )
  return mf(uid, iid, ue_c, ie_c, b_u, b_i)
